# baseline (speedup 1.0000x reference)
import functools

import jax
import jax.numpy as jnp
from jax import lax
from jax.experimental import pallas as pl
from jax.experimental.pallas import tpu as pltpu
from jax.experimental.pallas import tpu_sc as plsc

_NUM_BINS = 64
_N = 16777216
_NC = 2
_NS = 16
_L = 16
_NW = _NC * _NS
_PER_W = _N // _NW
_CHUNK = 32768
_NCHUNK = _PER_W // _CHUNK
_VPC = _CHUNK // _L
_S = 4
_UNROLL = 4
_HS = 8
_HUNROLL = 2

_mesh = plsc.VectorSubcoreMesh(core_axis_name="c", subcore_axis_name="s")


@functools.partial(
    pl.kernel,
    out_type=tuple(jax.ShapeDtypeStruct((_NW, _L), jnp.float32) for _ in range(4)),
    mesh=_mesh,
    compiler_params=pltpu.CompilerParams(needs_layout_passes=False),
    scratch_types=[
        pltpu.VMEM((_CHUNK,), jnp.float32),
        pltpu.VMEM((_CHUNK,), jnp.float32),
        pltpu.VMEM((_L,), jnp.float32),
        pltpu.VMEM((_L,), jnp.float32),
        pltpu.VMEM((_L,), jnp.float32),
        pltpu.VMEM((_L,), jnp.float32),
        pltpu.SemaphoreType.DMA,
        pltpu.SemaphoreType.DMA,
    ],
)
def _stats_kernel(arr, omin, omax, osum, oss,
                  buf0, buf1, smin, smax, ssum, sss, sem0, sem1):
    wid = lax.axis_index("c") * _NS + lax.axis_index("s")
    base = wid * _PER_W
    bufs = (buf0, buf1)
    sems = (sem0, sem1)

    copies = [None, None]
    copies[0] = pltpu.async_copy(arr.at[pl.ds(base, _CHUNK)], buf0, sem0)

    vmin = jnp.full((_L,), jnp.inf, jnp.float32)
    vmax = jnp.full((_L,), -jnp.inf, jnp.float32)
    vsum = jnp.zeros((_L,), jnp.float32)
    vss = jnp.zeros((_L,), jnp.float32)
    carry = tuple((vmin, vmax, vsum, vss) for _ in range(_S))

    for k in range(_NCHUNK):
        b = k % 2
        nb = (k + 1) % 2
        if k + 1 < _NCHUNK:
            copies[nb] = pltpu.async_copy(
                arr.at[pl.ds(base + (k + 1) * _CHUNK, _CHUNK)], bufs[nb], sems[nb])
        copies[b].wait()
        buf = bufs[b]

        def vstep(i, c, buf=buf):
            out = []
            for j in range(_S):
                mn, mx, s, ss = c[j]
                v = buf[pl.ds((i + j) * _L, _L)]
                out.append((jnp.minimum(mn, v), jnp.maximum(mx, v),
                            s + v, ss + v * v))
            return tuple(out)

        carry = plsc.parallel_loop(0, _VPC, _S, unroll=_UNROLL, carry=carry)(vstep)

    smin[...] = functools.reduce(jnp.minimum, [c[0] for c in carry])
    smax[...] = functools.reduce(jnp.maximum, [c[1] for c in carry])
    ssum[...] = functools.reduce(jnp.add, [c[2] for c in carry])
    sss[...] = functools.reduce(jnp.add, [c[3] for c in carry])
    pltpu.sync_copy(smin, omin.at[wid])
    pltpu.sync_copy(smax, omax.at[wid])
    pltpu.sync_copy(ssum, osum.at[wid])
    pltpu.sync_copy(sss, oss.at[wid])


@functools.partial(
    pl.kernel,
    out_type=jax.ShapeDtypeStruct((_NW, _NUM_BINS * _L), jnp.float32),
    mesh=_mesh,
    compiler_params=pltpu.CompilerParams(needs_layout_passes=False),
    scratch_types=[
        pltpu.VMEM((_CHUNK,), jnp.float32),
        pltpu.VMEM((_CHUNK,), jnp.float32),
        pltpu.VMEM((2, _L), jnp.float32),
        pltpu.VMEM((_NUM_BINS * _L,), jnp.float32),
        pltpu.SemaphoreType.DMA,
        pltpu.SemaphoreType.DMA,
    ],
)
def _hist_kernel(arr, params, ohist, buf0, buf1, pbuf, hist, sem0, sem1):
    wid = lax.axis_index("c") * _NS + lax.axis_index("s")
    base = wid * _PER_W
    bufs = (buf0, buf1)
    sems = (sem0, sem1)

    copies = [None, None]
    copies[0] = pltpu.async_copy(arr.at[pl.ds(base, _CHUNK)], buf0, sem0)
    pltpu.sync_copy(params, pbuf)
    scalev = pbuf[0, :]
    shiftv = pbuf[1, :]

    zero = jnp.zeros((_L,), jnp.float32)
    for j in range(_NUM_BINS):
        hist[pl.ds(j * _L, _L)] = zero

    lane64 = lax.broadcasted_iota(jnp.int32, (_L,), 0) * _NUM_BINS
    ones = jnp.ones((_L,), jnp.float32)
    top = jnp.full((_L,), _NUM_BINS - 1, jnp.int32)

    for k in range(_NCHUNK):
        b = k % 2
        nb = (k + 1) % 2
        if k + 1 < _NCHUNK:
            copies[nb] = pltpu.async_copy(
                arr.at[pl.ds(base + (k + 1) * _CHUNK, _CHUNK)], bufs[nb], sems[nb])
        copies[b].wait()
        buf = bufs[b]

        def vstep(i, buf=buf):
            for j in range(_HS):
                v = buf[pl.ds((i + j) * _L, _L)]
                t = v * scalev + shiftv
                bin_ = jnp.minimum(t.astype(jnp.int32), top)
                idx = bin_ + lane64
                plsc.addupdate_scatter(hist, [idx], ones)

        plsc.parallel_loop(0, _VPC, _HS, unroll=_HUNROLL)(vstep)

    pltpu.sync_copy(hist, ohist.at[wid])


def kernel(array):
    a = array.reshape(_N)
    mins, maxs, sums, sqs = _stats_kernel(a)
    mn = mins.min()
    mx = maxs.max()
    s = sums.sum()
    ss = sqs.sum()
    edges = jnp.linspace(mn, mx, _NUM_BINS + 1, dtype=jnp.float32)
    span = mx - mn
    ok = span > 0
    scale = jnp.where(ok, _NUM_BINS / span, 0.0).astype(jnp.float32)
    shift = jnp.where(ok, -mn * scale, jnp.float32(_NUM_BINS - 1))
    params = jnp.stack([jnp.full((_L,), scale, jnp.float32),
                        jnp.full((_L,), shift, jnp.float32)])
    hist = _hist_kernel(a, params)
    counts = hist.reshape(_NW, _L, _NUM_BINS).sum(axis=(0, 1))
    num = jnp.array(_N, dtype=jnp.int32)
    return (mn, mx, num, s, ss, edges, counts)


# speedup vs baseline: 1.0606x; 1.0606x over previous
import functools

import jax
import jax.numpy as jnp
from jax import lax
from jax.experimental import pallas as pl
from jax.experimental.pallas import tpu as pltpu
from jax.experimental.pallas import tpu_sc as plsc

_NUM_BINS = 64
_N = 16777216
_NC = 2
_NS = 16
_L = 16
_NW = _NC * _NS
_PER_W = _N // _NW
_CHUNK = 32768
_NCHUNK = _PER_W // _CHUNK
_VPC = _CHUNK // _L
_S = 4
_UNROLL = 2
_HS = 4
_HUNROLL = 2

_mesh = plsc.VectorSubcoreMesh(core_axis_name="c", subcore_axis_name="s")


@functools.partial(
    pl.kernel,
    out_type=tuple(jax.ShapeDtypeStruct((_NW, _L), jnp.float32) for _ in range(4)),
    mesh=_mesh,
    compiler_params=pltpu.CompilerParams(needs_layout_passes=False),
    scratch_types=[
        pltpu.VMEM((_CHUNK,), jnp.float32),
        pltpu.VMEM((_CHUNK,), jnp.float32),
        pltpu.VMEM((_L,), jnp.float32),
        pltpu.VMEM((_L,), jnp.float32),
        pltpu.VMEM((_L,), jnp.float32),
        pltpu.VMEM((_L,), jnp.float32),
        pltpu.SemaphoreType.DMA,
        pltpu.SemaphoreType.DMA,
    ],
)
def _stats_kernel(arr, omin, omax, osum, oss,
                  buf0, buf1, smin, smax, ssum, sss, sem0, sem1):
    wid = lax.axis_index("c") * _NS + lax.axis_index("s")
    base = wid * _PER_W
    bufs = (buf0, buf1)
    sems = (sem0, sem1)

    copies = [None, None]
    copies[0] = pltpu.async_copy(arr.at[pl.ds(base, _CHUNK)], buf0, sem0)

    vmin = jnp.full((_L,), jnp.inf, jnp.float32)
    vmax = jnp.full((_L,), -jnp.inf, jnp.float32)
    vsum = jnp.zeros((_L,), jnp.float32)
    vss = jnp.zeros((_L,), jnp.float32)
    carry = tuple((vmin, vmax, vsum, vss) for _ in range(_S))

    for k in range(_NCHUNK):
        b = k % 2
        nb = (k + 1) % 2
        if k + 1 < _NCHUNK:
            copies[nb] = pltpu.async_copy(
                arr.at[pl.ds(base + (k + 1) * _CHUNK, _CHUNK)], bufs[nb], sems[nb])
        copies[b].wait()
        buf = bufs[b]

        def vstep(i, c, buf=buf):
            out = []
            for j in range(_S):
                mn, mx, s, ss = c[j]
                v = buf[pl.ds((i + j) * _L, _L)]
                out.append((jnp.minimum(mn, v), jnp.maximum(mx, v),
                            s + v, ss + v * v))
            return tuple(out)

        carry = plsc.parallel_loop(0, _VPC, _S, unroll=_UNROLL, carry=carry)(vstep)

    smin[...] = functools.reduce(jnp.minimum, [c[0] for c in carry])
    smax[...] = functools.reduce(jnp.maximum, [c[1] for c in carry])
    ssum[...] = functools.reduce(jnp.add, [c[2] for c in carry])
    sss[...] = functools.reduce(jnp.add, [c[3] for c in carry])
    pltpu.sync_copy(smin, omin.at[wid])
    pltpu.sync_copy(smax, omax.at[wid])
    pltpu.sync_copy(ssum, osum.at[wid])
    pltpu.sync_copy(sss, oss.at[wid])


@functools.partial(
    pl.kernel,
    out_type=jax.ShapeDtypeStruct((_NW, _NUM_BINS * _L), jnp.float32),
    mesh=_mesh,
    compiler_params=pltpu.CompilerParams(needs_layout_passes=False),
    scratch_types=[
        pltpu.VMEM((_CHUNK,), jnp.float32),
        pltpu.VMEM((_CHUNK,), jnp.float32),
        pltpu.VMEM((2, _L), jnp.float32),
        pltpu.VMEM((_NUM_BINS * _L,), jnp.float32),
        pltpu.SemaphoreType.DMA,
        pltpu.SemaphoreType.DMA,
    ],
)
def _hist_kernel(arr, params, ohist, buf0, buf1, pbuf, hist, sem0, sem1):
    wid = lax.axis_index("c") * _NS + lax.axis_index("s")
    base = wid * _PER_W
    bufs = (buf0, buf1)
    sems = (sem0, sem1)

    copies = [None, None]
    copies[0] = pltpu.async_copy(arr.at[pl.ds(base, _CHUNK)], buf0, sem0)
    pltpu.sync_copy(params, pbuf)
    scalev = pbuf[0, :]
    shiftv = pbuf[1, :]

    zero = jnp.zeros((_L,), jnp.float32)
    for j in range(_NUM_BINS):
        hist[pl.ds(j * _L, _L)] = zero

    lane64 = lax.broadcasted_iota(jnp.int32, (_L,), 0) * _NUM_BINS
    ones = jnp.ones((_L,), jnp.float32)
    top = jnp.full((_L,), _NUM_BINS - 1, jnp.int32)

    for k in range(_NCHUNK):
        b = k % 2
        nb = (k + 1) % 2
        if k + 1 < _NCHUNK:
            copies[nb] = pltpu.async_copy(
                arr.at[pl.ds(base + (k + 1) * _CHUNK, _CHUNK)], bufs[nb], sems[nb])
        copies[b].wait()
        buf = bufs[b]

        def vstep(i, buf=buf):
            for j in range(_HS):
                v = buf[pl.ds((i + j) * _L, _L)]
                t = v * scalev + shiftv
                bin_ = jnp.minimum(t.astype(jnp.int32), top)
                idx = bin_ + lane64
                plsc.addupdate_scatter(hist, [idx], ones)

        plsc.parallel_loop(0, _VPC, _HS, unroll=_HUNROLL)(vstep)

    pltpu.sync_copy(hist, ohist.at[wid])


def kernel(array):
    a = array.reshape(_N)
    mins, maxs, sums, sqs = _stats_kernel(a)
    mn = mins.min()
    mx = maxs.max()
    s = sums.sum()
    ss = sqs.sum()
    edges = jnp.linspace(mn, mx, _NUM_BINS + 1, dtype=jnp.float32)
    span = mx - mn
    ok = span > 0
    scale = jnp.where(ok, _NUM_BINS / span, 0.0).astype(jnp.float32)
    shift = jnp.where(ok, -mn * scale, jnp.float32(_NUM_BINS - 1))
    params = jnp.stack([jnp.full((_L,), scale, jnp.float32),
                        jnp.full((_L,), shift, jnp.float32)])
    hist = _hist_kernel(a, params)
    counts = hist.reshape(_NW, _L, _NUM_BINS).sum(axis=(0, 1))
    num = jnp.array(_N, dtype=jnp.int32)
    return (mn, mx, num, s, ss, edges, counts)


# speedup vs baseline: 1.1207x; 1.0566x over previous
import functools

import jax
import jax.numpy as jnp
from jax import lax
from jax.experimental import pallas as pl
from jax.experimental.pallas import tpu as pltpu
from jax.experimental.pallas import tpu_sc as plsc

_NUM_BINS = 64
_N = 16777216
_NC = 2
_NS = 16
_L = 16
_NW = _NC * _NS
_PER_W = _N // _NW
_CHUNK = 32768
_NCHUNK = _PER_W // _CHUNK
_VPC = _CHUNK // _L
_S = 4
_UNROLL = 2
_HS = 4
_HUNROLL = 2
_NB = 2

_mesh = plsc.VectorSubcoreMesh(core_axis_name="c", subcore_axis_name="s")


@functools.partial(
    pl.kernel,
    out_type=tuple(jax.ShapeDtypeStruct((_NW, _L), jnp.float32) for _ in range(4)),
    mesh=_mesh,
    compiler_params=pltpu.CompilerParams(needs_layout_passes=False),
    scratch_types=[
        pltpu.VMEM((_CHUNK,), jnp.float32),
        pltpu.VMEM((_CHUNK,), jnp.float32),
        pltpu.VMEM((_L,), jnp.float32),
        pltpu.VMEM((_L,), jnp.float32),
        pltpu.VMEM((_L,), jnp.float32),
        pltpu.VMEM((_L,), jnp.float32),
        pltpu.SemaphoreType.DMA,
        pltpu.SemaphoreType.DMA,
    ],
)
def _stats_kernel(arr, omin, omax, osum, oss,
                  buf0, buf1, smin, smax, ssum, sss, sem0, sem1):
    wid = lax.axis_index("c") * _NS + lax.axis_index("s")
    base = wid * _PER_W
    bufs = (buf0, buf1)
    sems = (sem0, sem1)

    copies = [None, None]
    copies[0] = pltpu.async_copy(arr.at[pl.ds(base, _CHUNK)], buf0, sem0)

    vmin = jnp.full((_L,), jnp.inf, jnp.float32)
    vmax = jnp.full((_L,), -jnp.inf, jnp.float32)
    vsum = jnp.zeros((_L,), jnp.float32)
    vss = jnp.zeros((_L,), jnp.float32)
    carry = tuple((vmin, vmax, vsum, vss) for _ in range(_S))

    for k in range(_NCHUNK):
        b = k % 2
        nb = (k + 1) % 2
        if k + 1 < _NCHUNK:
            copies[nb] = pltpu.async_copy(
                arr.at[pl.ds(base + (k + 1) * _CHUNK, _CHUNK)], bufs[nb], sems[nb])
        copies[b].wait()
        buf = bufs[b]

        def vstep(i, c, buf=buf):
            out = []
            for j in range(_S):
                mn, mx, s, ss = c[j]
                v = buf[pl.ds((i + j) * _L, _L)]
                out.append((jnp.minimum(mn, v), jnp.maximum(mx, v),
                            s + v, ss + v * v))
            return tuple(out)

        carry = plsc.parallel_loop(0, _VPC, _S, unroll=_UNROLL, carry=carry)(vstep)

    smin[...] = functools.reduce(jnp.minimum, [c[0] for c in carry])
    smax[...] = functools.reduce(jnp.maximum, [c[1] for c in carry])
    ssum[...] = functools.reduce(jnp.add, [c[2] for c in carry])
    sss[...] = functools.reduce(jnp.add, [c[3] for c in carry])
    pltpu.sync_copy(smin, omin.at[wid])
    pltpu.sync_copy(smax, omax.at[wid])
    pltpu.sync_copy(ssum, osum.at[wid])
    pltpu.sync_copy(sss, oss.at[wid])


@functools.partial(
    pl.kernel,
    out_type=jax.ShapeDtypeStruct((_NW, _NB * _NUM_BINS * _L), jnp.float32),
    mesh=_mesh,
    compiler_params=pltpu.CompilerParams(needs_layout_passes=False),
    scratch_types=[
        pltpu.VMEM((_CHUNK,), jnp.float32),
        pltpu.VMEM((_CHUNK,), jnp.float32),
        pltpu.VMEM((2, _L), jnp.float32),
        pltpu.VMEM((_NB * _NUM_BINS * _L,), jnp.float32),
        pltpu.SemaphoreType.DMA,
        pltpu.SemaphoreType.DMA,
    ],
)
def _hist_kernel(arr, params, ohist, buf0, buf1, pbuf, hist, sem0, sem1):
    wid = lax.axis_index("c") * _NS + lax.axis_index("s")
    base = wid * _PER_W
    bufs = (buf0, buf1)
    sems = (sem0, sem1)

    copies = [None, None]
    copies[0] = pltpu.async_copy(arr.at[pl.ds(base, _CHUNK)], buf0, sem0)
    pltpu.sync_copy(params, pbuf)
    scalev = pbuf[0, :]
    shiftv = pbuf[1, :]

    zero = jnp.zeros((_L,), jnp.float32)
    for j in range(_NB * _NUM_BINS):
        hist[pl.ds(j * _L, _L)] = zero

    lane = lax.broadcasted_iota(jnp.int32, (_L,), 0)
    ones = jnp.ones((_L,), jnp.float32)
    top = jnp.full((_L,), _NUM_BINS - 1, jnp.int32)

    for k in range(_NCHUNK):
        b = k % 2
        nb = (k + 1) % 2
        if k + 1 < _NCHUNK:
            copies[nb] = pltpu.async_copy(
                arr.at[pl.ds(base + (k + 1) * _CHUNK, _CHUNK)], bufs[nb], sems[nb])
        copies[b].wait()
        buf = bufs[b]

        def vstep(i, buf=buf):
            for j in range(_HS):
                v = buf[pl.ds((i + j) * _L, _L)]
                t = v * scalev + shiftv
                bin_ = jnp.minimum(t.astype(jnp.int32), top)
                idx = bin_ * _L + (lane + (j % _NB) * (_NUM_BINS * _L))
                plsc.addupdate_scatter(hist, [idx], ones)

        plsc.parallel_loop(0, _VPC, _HS, unroll=_HUNROLL)(vstep)

    pltpu.sync_copy(hist, ohist.at[wid])


def kernel(array):
    a = array.reshape(_N)
    mins, maxs, sums, sqs = _stats_kernel(a)
    mn = mins.min()
    mx = maxs.max()
    s = sums.sum()
    ss = sqs.sum()
    edges = jnp.linspace(mn, mx, _NUM_BINS + 1, dtype=jnp.float32)
    span = mx - mn
    ok = span > 0
    scale = jnp.where(ok, _NUM_BINS / span, 0.0).astype(jnp.float32)
    shift = jnp.where(ok, -mn * scale, jnp.float32(_NUM_BINS - 1))
    params = jnp.stack([jnp.full((_L,), scale, jnp.float32),
                        jnp.full((_L,), shift, jnp.float32)])
    hist = _hist_kernel(a, params)
    counts = hist.reshape(_NW * _NB, _NUM_BINS, _L).sum(axis=(0, 2))
    num = jnp.array(_N, dtype=jnp.int32)
    return (mn, mx, num, s, ss, edges, counts)


# speedup vs baseline: 1.3917x; 1.2418x over previous
import functools

import jax
import jax.numpy as jnp
import numpy as np
from jax import lax
from jax.experimental import pallas as pl
from jax.experimental.pallas import tpu as pltpu
from jax.experimental.pallas import tpu_sc as plsc

_NUM_BINS = 64
_N = 16777216
_NC = 2
_NS = 16
_L = 16
_NW = _NC * _NS
_PER_W = _N // _NW
_CHUNK = 32768
_NCHUNK = _PER_W // _CHUNK
_VPC = _CHUNK // _L
_S = 4
_UNROLL = 2
_HS = 4
_HUNROLL = 2
_NB = 1
_MAGIC = 12582912.0
_MAGIC_BITS = 0x4B400000

_mesh = plsc.VectorSubcoreMesh(core_axis_name="c", subcore_axis_name="s")


@functools.partial(
    pl.kernel,
    out_type=tuple(jax.ShapeDtypeStruct((_NW, _L), jnp.float32) for _ in range(4)),
    mesh=_mesh,
    compiler_params=pltpu.CompilerParams(needs_layout_passes=False),
    scratch_types=[
        pltpu.VMEM((_CHUNK,), jnp.float32),
        pltpu.VMEM((_CHUNK,), jnp.float32),
        pltpu.VMEM((_L,), jnp.float32),
        pltpu.VMEM((_L,), jnp.float32),
        pltpu.VMEM((_L,), jnp.float32),
        pltpu.VMEM((_L,), jnp.float32),
        pltpu.SemaphoreType.DMA,
        pltpu.SemaphoreType.DMA,
    ],
)
def _stats_kernel(arr, omin, omax, osum, oss,
                  buf0, buf1, smin, smax, ssum, sss, sem0, sem1):
    wid = lax.axis_index("c") * _NS + lax.axis_index("s")
    base = wid * _PER_W
    bufs = (buf0, buf1)
    sems = (sem0, sem1)

    copies = [None, None]
    copies[0] = pltpu.async_copy(arr.at[pl.ds(base, _CHUNK)], buf0, sem0)

    vmin = jnp.full((_L,), jnp.inf, jnp.float32)
    vmax = jnp.full((_L,), -jnp.inf, jnp.float32)
    vsum = jnp.zeros((_L,), jnp.float32)
    vss = jnp.zeros((_L,), jnp.float32)
    carry = tuple((vmin, vmax, vsum, vss) for _ in range(_S))

    for k in range(_NCHUNK):
        b = k % 2
        nb = (k + 1) % 2
        if k + 1 < _NCHUNK:
            copies[nb] = pltpu.async_copy(
                arr.at[pl.ds(base + (k + 1) * _CHUNK, _CHUNK)], bufs[nb], sems[nb])
        copies[b].wait()
        buf = bufs[b]

        def vstep(i, c, buf=buf):
            out = []
            for j in range(_S):
                mn, mx, s, ss = c[j]
                v = buf[pl.ds((i + j) * _L, _L)]
                out.append((jnp.minimum(mn, v), jnp.maximum(mx, v),
                            s + v, ss + v * v))
            return tuple(out)

        carry = plsc.parallel_loop(0, _VPC, _S, unroll=_UNROLL, carry=carry)(vstep)

    smin[...] = functools.reduce(jnp.minimum, [c[0] for c in carry])
    smax[...] = functools.reduce(jnp.maximum, [c[1] for c in carry])
    ssum[...] = functools.reduce(jnp.add, [c[2] for c in carry])
    sss[...] = functools.reduce(jnp.add, [c[3] for c in carry])
    pltpu.sync_copy(smin, omin.at[wid])
    pltpu.sync_copy(smax, omax.at[wid])
    pltpu.sync_copy(ssum, osum.at[wid])
    pltpu.sync_copy(sss, oss.at[wid])


@functools.partial(
    pl.kernel,
    out_type=jax.ShapeDtypeStruct((_NW, _NB * _NUM_BINS * _L), jnp.float32),
    mesh=_mesh,
    compiler_params=pltpu.CompilerParams(needs_layout_passes=False),
    scratch_types=[
        pltpu.VMEM((_CHUNK,), jnp.float32),
        pltpu.VMEM((_CHUNK,), jnp.float32),
        pltpu.VMEM((2, _L), jnp.float32),
        pltpu.VMEM((_NB * _NUM_BINS * _L,), jnp.float32),
        pltpu.SemaphoreType.DMA,
        pltpu.SemaphoreType.DMA,
    ],
)
def _hist_kernel(arr, params, ohist, buf0, buf1, pbuf, hist, sem0, sem1):
    wid = lax.axis_index("c") * _NS + lax.axis_index("s")
    base = wid * _PER_W
    bufs = (buf0, buf1)
    sems = (sem0, sem1)

    copies = [None, None]
    copies[0] = pltpu.async_copy(arr.at[pl.ds(base, _CHUNK)], buf0, sem0)
    pltpu.sync_copy(params, pbuf)
    scalev = pbuf[0, :]
    shiftv = pbuf[1, :]

    zero = jnp.zeros((_L,), jnp.float32)
    for j in range(_NB * _NUM_BINS):
        hist[pl.ds(j * _L, _L)] = zero

    lane = lax.broadcasted_iota(jnp.int32, (_L,), 0)
    ones = jnp.ones((_L,), jnp.float32)
    magic = jnp.full((_L,), _MAGIC, jnp.float32)
    _base = (_MAGIC_BITS << 4) & 0xFFFFFFFF
    lane_adj = lane + jnp.int32(np.uint32(0x100000000 - _base).astype(np.int32))
    cap = jnp.full((_L,), _NB * _NUM_BINS * _L - 1, jnp.uint32)

    for k in range(_NCHUNK):
        b = k % 2
        nb = (k + 1) % 2
        if k + 1 < _NCHUNK:
            copies[nb] = pltpu.async_copy(
                arr.at[pl.ds(base + (k + 1) * _CHUNK, _CHUNK)], bufs[nb], sems[nb])
        copies[b].wait()
        buf = bufs[b]

        def vstep(i, buf=buf):
            for j in range(_HS):
                v = buf[pl.ds((i + j) * _L, _L)]
                t = v * scalev + shiftv
                m = t + magic
                bits = plsc.bitcast(m, jnp.int32)
                idx = jnp.left_shift(bits, 4) + lane_adj
                idxu = jnp.minimum(plsc.bitcast(idx, jnp.uint32), cap)
                plsc.addupdate_scatter(hist, [plsc.bitcast(idxu, jnp.int32)], ones)

        plsc.parallel_loop(0, _VPC, _HS, unroll=_HUNROLL)(vstep)

    pltpu.sync_copy(hist, ohist.at[wid])


def kernel(array):
    a = array.reshape(_N)
    mins, maxs, sums, sqs = _stats_kernel(a)
    mn = mins.min()
    mx = maxs.max()
    s = sums.sum()
    ss = sqs.sum()
    edges = jnp.linspace(mn, mx, _NUM_BINS + 1, dtype=jnp.float32)
    span = mx - mn
    ok = span > 0
    scale = jnp.where(ok, _NUM_BINS * (1.0 - 2.0 ** -19) / span, 0.0)
    scale = scale.astype(jnp.float32)
    shift = jnp.where(ok, -mn * scale - 0.5, jnp.float32(_NUM_BINS - 0.75))
    params = jnp.stack([jnp.full((_L,), scale, jnp.float32),
                        jnp.full((_L,), shift, jnp.float32)])
    hist = _hist_kernel(a, params)
    counts = hist.reshape(_NW * _NB, _NUM_BINS, _L).sum(axis=(0, 2))
    num = jnp.array(_N, dtype=jnp.int32)
    return (mn, mx, num, s, ss, edges, counts)
